# baseline (speedup 1.0000x reference)
import functools

import jax
import jax.numpy as jnp
from jax import lax
from jax.experimental import pallas as pl
from jax.experimental.pallas import tpu as pltpu
from jax.experimental.pallas import tpu_sc as plsc

EMB = 128
L = 16
HALF_ROWS = 128
BLOCKS_PER_HALF = HALF_ROWS // L


@functools.lru_cache(maxsize=None)
def _build(rows_per_worker: int):
    info = plsc.get_sparse_core_info()
    nc, ns = info.num_cores, info.num_subcores
    nw = nc * ns
    total_rows = nw * rows_per_worker
    n_halves = rows_per_worker // HALF_ROWS

    mesh = plsc.VectorSubcoreMesh(core_axis_name="c", subcore_axis_name="s")

    @functools.partial(
        pl.kernel,
        mesh=mesh,
        compiler_params=pltpu.CompilerParams(needs_layout_passes=False),
        out_type=jax.ShapeDtypeStruct((total_rows * EMB,), jnp.float32),
        scratch_types=[
            pltpu.VMEM((n_halves, HALF_ROWS), jnp.int32),
            pltpu.VMEM((EMB * EMB,), jnp.float32),
            pltpu.VMEM((2 * HALF_ROWS * EMB,), jnp.float32),
            pltpu.SemaphoreType.DMA,
        ],
    )
    def k(idx_hbm, table_hbm, out_hbm, idx_v, tab_v, stage_v, wsem):
        sid = lax.axis_index("s")
        wid = sid * nc + lax.axis_index("c")
        base = wid * rows_per_worker * EMB

        pltpu.sync_copy(table_hbm, tab_v)
        pltpu.sync_copy(idx_hbm.at[wid], idx_v)

        lane = lax.iota(jnp.int32, L)
        one = lax.full((L,), jnp.int32(1), jnp.int32)

        def write(h, o):
            return pltpu.make_async_copy(
                stage_v.at[pl.ds(h * HALF_ROWS * EMB, HALF_ROWS * EMB)],
                out_hbm.at[pl.ds(base + o * HALF_ROWS * EMB, HALF_ROWS * EMB)],
                wsem,
            )

        def route_half(h, o):
            def block(k2, carry):
                rows16 = idx_v[o, pl.ds(k2 * L, L)]
                a = rows16 * EMB
                d = lane * EMB + (h * HALF_ROWS + k2 * L) * EMB
                for _ in range(EMB):
                    v = plsc.load_gather(tab_v, [a])
                    plsc.store_scatter(stage_v, [d], v)
                    a = a + one
                    d = d + one
                return carry

            lax.fori_loop(0, BLOCKS_PER_HALF, block, 0, unroll=False)

        def group(g, carry):
            for h in range(2):
                o = g * 2 + h
                @pl.when(o >= 2)
                def _():
                    write(0, 0).wait()

                route_half(h, o)
                write(h, o).start()

            return carry

        lax.fori_loop(0, n_halves // 2, group, 0, unroll=False)
        write(0, 0).wait()
        write(0, 0).wait()

    return k


def kernel(board, W):
    bsz, seq = board.shape
    total = bsz * seq
    info = plsc.get_sparse_core_info()
    nw = info.num_cores * info.num_subcores
    rows_per_worker = total // nw
    idx = board.reshape(nw, rows_per_worker // HALF_ROWS, HALF_ROWS).astype(
        jnp.int32
    )
    out = _build(rows_per_worker)(idx, W.reshape(-1))
    return out.reshape(bsz, seq, EMB)


# speedup vs baseline: 20.5764x; 20.5764x over previous
import functools

import jax
import jax.numpy as jnp
from jax import lax
from jax.experimental import pallas as pl
from jax.experimental.pallas import tpu as pltpu
from jax.experimental.pallas import tpu_sc as plsc

EMB = 128
ROWS_PER_OP = 128
NB = 5


@functools.lru_cache(maxsize=None)
def _build(n_ops_per_worker: int):
    info = plsc.get_sparse_core_info()
    nc, ns = info.num_cores, info.num_subcores
    nw = nc * ns
    rows_per_worker = n_ops_per_worker * ROWS_PER_OP
    total_rows = nw * rows_per_worker

    mesh = plsc.VectorSubcoreMesh(core_axis_name="c", subcore_axis_name="s")

    @functools.partial(
        pl.kernel,
        mesh=mesh,
        out_type=jax.ShapeDtypeStruct((total_rows, EMB), jnp.float32),
        scratch_types=[
            pltpu.VMEM((n_ops_per_worker, ROWS_PER_OP), jnp.int32),
            pltpu.VMEM((NB, ROWS_PER_OP, EMB), jnp.float32),
            pltpu.VMEM_SHARED((EMB, EMB), jnp.float32),
            pltpu.SemaphoreType.DMA,
        ],
    )
    def k(idx_hbm, table_hbm, out_hbm, idx_v, rows_v, table_sp, gsem):
        sid = lax.axis_index("s")
        wid = sid * nc + lax.axis_index("c")
        base = wid * rows_per_worker

        @pl.when(sid == 0)
        def _():
            pltpu.sync_copy(table_hbm, table_sp)

        pltpu.sync_copy(idx_hbm.at[wid], idx_v)
        plsc.subcore_barrier()

        for b in range(NB):
            pltpu.async_copy(table_sp.at[idx_v.at[b]], rows_v.at[b], gsem)

        def group(g, carry):
            for b in range(NB):
                j = g * NB + b
                pltpu.make_async_copy(
                    table_sp.at[idx_v.at[b]], rows_v.at[b], gsem
                ).wait()
                pltpu.sync_copy(
                    rows_v.at[b],
                    out_hbm.at[pl.ds(base + j * ROWS_PER_OP, ROWS_PER_OP)],
                )
                nj = j + NB

                @pl.when(nj < n_ops_per_worker)
                def _():
                    pltpu.async_copy(
                        table_sp.at[idx_v.at[nj]], rows_v.at[b], gsem
                    )

            return carry

        lax.fori_loop(0, n_ops_per_worker // NB, group, 0, unroll=False)

    return k


def kernel(board, W):
    bsz, seq = board.shape
    total = bsz * seq
    info = plsc.get_sparse_core_info()
    nw = info.num_cores * info.num_subcores
    n_ops = total // (nw * ROWS_PER_OP)
    idx = board.reshape(nw, n_ops, ROWS_PER_OP).astype(jnp.int32)
    out = _build(n_ops)(idx, W)
    return out.reshape(bsz, seq, EMB)
